# Initial kernel scaffold; baseline (speedup 1.0000x reference)
import jax
import jax.numpy as jnp
from jax.experimental import pallas as pl


def kernel(x, tables, Wc, bc):
    raise NotImplementedError("write your pallas kernel here")


# speedup vs baseline: 1.1365x; 1.1365x over previous
import functools

import jax
import jax.numpy as jnp
from jax import lax
from jax.experimental import pallas as pl
from jax.experimental.pallas import tpu as pltpu
from jax.experimental.pallas import tpu_sc as plsc

_VOCAB = 100000
_EMB = 32
_NCAT = 20
_NCONT = 6
_NFEAT = _NCAT + _NCONT

_NW = 32
_CH = 64
_KG = _CH * _NCAT // 128
_KC = _CH * _NCONT // 128


def _silu(v):
    return v / (1.0 + jnp.exp(-v))


def _build(bt):
    pos_per_w = bt // _NW
    n_chunks = pos_per_w // _CH
    mesh = plsc.VectorSubcoreMesh(core_axis_name="c", subcore_axis_name="s")

    @functools.partial(
        pl.kernel,
        mesh=mesh,
        compiler_params=pltpu.CompilerParams(
            use_tc_tiling_on_sc=False, needs_layout_passes=False),
        out_type=jax.ShapeDtypeStruct((bt * _NFEAT, _EMB), jnp.float32),
        scratch_types=[
            pltpu.VMEM((_CH * _NFEAT,), jnp.float32),
            pltpu.VMEM((_CH * _NCAT,), jnp.int32),
            pltpu.VMEM((_KG, 128), jnp.int32),
            pltpu.VMEM((_KC, 128), jnp.int32),
            pltpu.VMEM((_CH * _NCAT, _EMB), jnp.float32),
            pltpu.VMEM((_CH * _NCONT, _EMB), jnp.float32),
            pltpu.VMEM((_NCONT * _EMB,), jnp.float32),
            pltpu.VMEM((_NCONT * _EMB,), jnp.float32),
            pltpu.SemaphoreType.DMA,
            pltpu.SemaphoreType.DMA,
        ],
    )
    def k(x_hbm, table_hbm, wc_hbm, bc_hbm, out_hbm,
          xv, idxv, dstv, cdstv, rows, contv, wcv, bcv, gsem, ssem):
        wid = lax.axis_index("s") * 2 + lax.axis_index("c")
        pltpu.sync_copy(wc_hbm, wcv)
        pltpu.sync_copy(bc_hbm, bcv)
        base_pos = wid * pos_per_w
        iota16 = lax.broadcasted_iota(jnp.int32, (16,), 0)

        def chunk_body(c, carry):
            p0 = base_pos + c * _CH
            o0 = p0 * _NFEAT
            pltpu.sync_copy(x_hbm.at[pl.ds(o0, _CH * _NFEAT)], xv)

            def blk_body(blk, _):
                i = iota16 + blk * 16
                d = (i.astype(jnp.float32) * (1.0 / _NCAT)).astype(jnp.int32)
                m = i - d * _NCAT
                src = i + (_NFEAT - _NCAT) * d
                xval = plsc.load_gather(xv, [src])
                idxv[pl.ds(blk * 16, 16)] = xval.astype(jnp.int32) + m * _VOCAB
                dstv[blk // 8, pl.ds((blk % 8) * 16, 16)] = src + o0
                return 0

            lax.fori_loop(0, _CH * _NCAT // 16, blk_body, 0)

            def cblk_body(blk, _):
                r = iota16 + blk * 16
                d = (r.astype(jnp.float32) * (1.0 / _NCONT)).astype(jnp.int32)
                cdstv[blk // 8, pl.ds((blk % 8) * 16, 16)] = (
                    o0 + _NCAT + r + _NCAT * d)
                return 0

            lax.fori_loop(0, _CH * _NCONT // 16, cblk_body, 0)

            handles = [
                pltpu.async_copy(
                    table_hbm.at[idxv.at[pl.ds(g * 128, 128)]],
                    rows.at[pl.ds(g * 128, 128)],
                    gsem,
                )
                for g in range(_KG)
            ]

            for f in range(_NCONT):
                wrows = [wcv[pl.ds(f * _EMB + h * 16, 16)] for h in range(2)]
                brows = [bcv[pl.ds(f * _EMB + h * 16, 16)] for h in range(2)]

                def pb_body(pb, _, f=f, wrows=wrows, brows=brows):
                    vec = plsc.load_gather(
                        xv, [iota16 * _NFEAT + (pb * 16 * _NFEAT + _NCAT + f)])
                    ridx = iota16 * _NCONT + (pb * 16 * _NCONT + f)
                    for e in range(_EMB):
                        w = wrows[e // 16][e % 16]
                        b = brows[e // 16][e % 16]
                        y = _silu(vec * w + b)
                        cidx = jnp.full((16,), e, jnp.int32)
                        plsc.store_scatter(contv, [ridx, cidx], y)
                    return 0

                lax.fori_loop(0, _CH // 16, pb_body, 0)

            for h in handles:
                h.wait()

            sh = [
                pltpu.async_copy(
                    rows.at[pl.ds(g * 128, 128)],
                    out_hbm.at[dstv.at[g]],
                    ssem,
                )
                for g in range(_KG)
            ] + [
                pltpu.async_copy(
                    contv.at[pl.ds(g * 128, 128)],
                    out_hbm.at[cdstv.at[g]],
                    ssem,
                )
                for g in range(_KC)
            ]
            for h in sh:
                h.wait()
            return carry

        lax.fori_loop(0, n_chunks, chunk_body, 0)

    return k


def kernel(x, tables, Wc, bc):
    B, T, _ = x.shape
    ncat, vocab, emb = tables.shape
    bt = B * T
    x1d = x.reshape(bt * _NFEAT)
    table2d = tables.reshape(ncat * vocab, emb)
    out2d = _build(bt)(x1d, table2d, Wc.reshape(-1), bc.reshape(-1))
    return out2d.reshape(B, T, _NFEAT, emb)
